# baseline (speedup 1.0000x reference)
import functools

import jax
import jax.numpy as jnp
from jax import lax
from jax.experimental import pallas as pl
from jax.experimental.pallas import tpu as pltpu
from jax.experimental.pallas import tpu_sc as plsc

NC = 2
NS = 16
NW = NC * NS
L = 16
CH = 128
D = 128

_EDGE_PAD = 2 * CH + 8


def _pad_edges(elist, value):
    z = jnp.zeros((_EDGE_PAD,), jnp.int32)
    zf = jnp.zeros((_EDGE_PAD,), jnp.float32)
    return jnp.concatenate([elist, z]), jnp.concatenate([value, zf])


def _vlb(vlist, n_rows, rpw):
    wb = ((rpw + 1 + 15) // 16) * 16
    idx = jnp.minimum(
        jnp.arange(NW, dtype=jnp.int32)[:, None] * rpw
        + jnp.arange(wb, dtype=jnp.int32)[None, :],
        n_rows,
    )
    return jnp.take(vlist, idx, axis=0).reshape(-1)


_GDN = lax.GatherDimensionNumbers(
    offset_dims=(), collapsed_slice_dims=(0,), start_index_map=(0,))


def _vbroadcast(v16, i):
    isp = jnp.full((L, 1), i, jnp.int32)
    return lax.gather(v16, isp, _GDN, (1,),
                      mode=lax.GatherScatterMode.PROMISE_IN_BOUNDS)


def _scalar(ref, i):
    v = ref[pl.ds((i // L) * L, L)]
    return v[i % L]


def _phase_runner(rpw, wb, nsteps, refs, wid, col_iota):
    (acc_v, gA, gB, vlb_v, idxA, idxB, valA, valB, rbA, rbB,
     semA, semB, semGA, semGB) = refs

    def search_chunk(base, e0, e1, val_v, rb_v):
        for g in range(CH // L):
            eid = jnp.full((L,), base + g * L, jnp.int32) + col_iota
            valid = (eid >= e0) & (eid < e1)
            v16 = val_v[pl.ds(g * L, L)]
            val_v[pl.ds(g * L, L)] = jnp.where(valid, v16, 0.0)
            lo = jnp.zeros((L,), jnp.int32)
            hi = jnp.full((L,), rpw, jnp.int32)
            for _s in range(nsteps):
                mid = (lo + hi) >> 1
                vm = plsc.load_gather(vlb_v, [mid])
                pred = vm <= eid
                lo = jnp.where(pred, mid, lo)
                hi = jnp.where(pred, hi, mid)
            rb_v[pl.ds(g * L, L)] = lo

    def edge_loop(g_v, val_v, rb_v):
        @plsc.parallel_loop(0, CH // L, unroll=2)
        def _(g):
            gb = g * L
            v16 = val_v[pl.ds(gb, L)]
            r16 = rb_v[pl.ds(gb, L)]
            for i in range(L):
                vj = _vbroadcast(v16, i)
                rj = _vbroadcast(r16, i)
                jsp = jnp.full((L,), gb + i, jnp.int32)
                for cb in range(D // L):
                    ci = col_iota + (cb * L)
                    g16 = plsc.load_gather(g_v, [jsp, ci])
                    plsc.addupdate_scatter(acc_v, [rj, ci], vj * g16)

    def run_phase(vlb_hbm, el_hbm, val_hbm, src_hbm):
        pltpu.sync_copy(vlb_hbm.at[pl.ds(pl.multiple_of(wid * wb, 8), wb)],
                        vlb_v)
        e0 = _scalar(vlb_v, 0)
        e1 = _scalar(vlb_v, rpw)
        e0a = (e0 >> 3) << 3
        npair = (e1 - e0a + 2 * CH - 1) // (2 * CH)

        def pair_body(k, _):
            baseA = pl.multiple_of(e0a + k * (2 * CH), 8)
            baseB = pl.multiple_of(baseA + CH, 8)
            cA1 = pltpu.async_copy(el_hbm.at[pl.ds(baseA, CH)], idxA, semA)
            cA2 = pltpu.async_copy(val_hbm.at[pl.ds(baseA, CH)], valA, semA)
            cB1 = pltpu.async_copy(el_hbm.at[pl.ds(baseB, CH)], idxB, semB)
            cB2 = pltpu.async_copy(val_hbm.at[pl.ds(baseB, CH)], valB, semB)
            cA1.wait()
            cA2.wait()
            gcA = pltpu.async_copy(src_hbm.at[idxA], gA, semGA)
            search_chunk(baseA, e0, e1, valA, rbA)
            cB1.wait()
            cB2.wait()
            gcB = pltpu.async_copy(src_hbm.at[idxB], gB, semGB)
            search_chunk(baseB, e0, e1, valB, rbB)
            gcA.wait()
            edge_loop(gA, valA, rbA)
            gcB.wait()
            edge_loop(gB, valB, rbB)
            return 0

        lax.fori_loop(0, npair, pair_body, 0)

    return run_phase


def _scratch(rpw, wb):
    return [
        pltpu.VMEM((rpw, D), jnp.float32),
        pltpu.VMEM((CH, D), jnp.float32),
        pltpu.VMEM((CH, D), jnp.float32),
        pltpu.VMEM((wb,), jnp.int32),
        pltpu.VMEM((CH,), jnp.int32),
        pltpu.VMEM((CH,), jnp.int32),
        pltpu.VMEM((CH,), jnp.float32),
        pltpu.VMEM((CH,), jnp.float32),
        pltpu.VMEM((CH,), jnp.int32),
        pltpu.VMEM((CH,), jnp.int32),
        pltpu.SemaphoreType.DMA,
        pltpu.SemaphoreType.DMA,
        pltpu.SemaphoreType.DMA,
        pltpu.SemaphoreType.DMA,
    ]


_MESH = dict(core_axis_name="c", subcore_axis_name="s",
             num_cores=NC, num_subcores=NS)
_CP = pltpu.CompilerParams(needs_layout_passes=False)


@functools.partial(jax.jit, static_argnames=("rpw",))
def _spmm_single(accum, vlb_a, elist_a, value_a, src_a, *, rpw):
    n_rows_pad = NW * rpw
    wb = vlb_a.shape[0] // NW
    nsteps = max(1, (rpw - 1).bit_length())

    def body(accum_hbm, vlba_hbm, ela_hbm, vala_hbm, srca_hbm, out_hbm,
             *refs):
        wid = lax.axis_index("c") * NS + lax.axis_index("s")
        r0 = wid * rpw
        col_iota = lax.iota(jnp.int32, L)
        acc_v = refs[0]
        run = _phase_runner(rpw, wb, nsteps, refs, wid, col_iota)
        pltpu.sync_copy(accum_hbm.at[pl.ds(r0, rpw)], acc_v)
        run(vlba_hbm, ela_hbm, vala_hbm, srca_hbm)
        pltpu.sync_copy(acc_v, out_hbm.at[pl.ds(r0, rpw)])

    f = pl.kernel(
        body,
        out_type=jax.ShapeDtypeStruct((n_rows_pad, D), jnp.float32),
        mesh=plsc.VectorSubcoreMesh(**_MESH),
        scratch_types=_scratch(rpw, wb),
        compiler_params=_CP,
    )
    return f(accum, vlb_a, elist_a, value_a, src_a)


@functools.partial(jax.jit, static_argnames=("rpw",))
def _spmm_double(accum, vlb_a, elist_a, value_a, src_a,
                 vlb_b, elist_b, value_b, src_b, *, rpw):
    n_rows_pad = NW * rpw
    wb = vlb_a.shape[0] // NW
    nsteps = max(1, (rpw - 1).bit_length())

    def body(accum_hbm, vlba_hbm, ela_hbm, vala_hbm, srca_hbm,
             vlbb_hbm, elb_hbm, valb_hbm, srcb_hbm, out_hbm, *refs):
        wid = lax.axis_index("c") * NS + lax.axis_index("s")
        r0 = wid * rpw
        col_iota = lax.iota(jnp.int32, L)
        acc_v = refs[0]
        run = _phase_runner(rpw, wb, nsteps, refs, wid, col_iota)
        pltpu.sync_copy(accum_hbm.at[pl.ds(r0, rpw)], acc_v)
        run(vlba_hbm, ela_hbm, vala_hbm, srca_hbm)
        run(vlbb_hbm, elb_hbm, valb_hbm, srcb_hbm)
        pltpu.sync_copy(acc_v, out_hbm.at[pl.ds(r0, rpw)])

    f = pl.kernel(
        body,
        out_type=jax.ShapeDtypeStruct((n_rows_pad, D), jnp.float32),
        mesh=plsc.VectorSubcoreMesh(**_MESH),
        scratch_types=_scratch(rpw, wb),
        compiler_params=_CP,
    )
    return f(accum, vlb_a, elist_a, value_a, src_a,
             vlb_b, elist_b, value_b, src_b)


def kernel(x_j, v2v_vlist, v2v_elist, v2v_value,
           r2v_vlist, r2v_elist, r2v_value,
           r2r0_vlist, r2r0_elist, r2r0_value,
           r2r1_vlist, r2r1_elist, r2r1_value,
           v2r_vlist, v2r_elist, v2r_value):
    N = x_j.shape[0]
    R = r2v_vlist.shape[0] - 1

    rpw_r = -(-((R + NW - 1) // NW) // 8) * 8
    rpw_n = -(-((N + NW - 1) // NW) // 8) * 8
    rp_r = NW * rpw_r
    rp_n = NW * rpw_n

    r2v_e, r2v_v = _pad_edges(r2v_elist, r2v_value)
    r2r0_e, r2r0_v = _pad_edges(r2r0_elist, r2r0_value)
    r2r1_e, r2r1_v = _pad_edges(r2r1_elist, r2r1_value)
    v2r_e, v2r_v = _pad_edges(v2r_elist, v2r_value)
    v2v_e, v2v_v = _pad_edges(v2v_elist, v2v_value)

    vlb_r2v = _vlb(r2v_vlist, R, rpw_r)
    vlb_r2r0 = _vlb(r2r0_vlist, R, rpw_r)
    vlb_r2r1 = _vlb(r2r1_vlist, R, rpw_r)
    vlb_v2r = _vlb(v2r_vlist, N, rpw_n)
    vlb_v2v = _vlb(v2v_vlist, N, rpw_n)

    zeros_r = jnp.zeros((rp_r, D), jnp.float32)
    zeros_n = jnp.zeros((rp_n, D), jnp.float32)

    rule0 = _spmm_single(zeros_r, vlb_r2v, r2v_e, r2v_v, x_j, rpw=rpw_r)
    rule1 = _spmm_single(rule0, vlb_r2r0, r2r0_e, r2r0_v, rule0, rpw=rpw_r)
    rule2 = _spmm_single(rule1, vlb_r2r1, r2r1_e, r2r1_v, rule1, rpw=rpw_r)
    out = _spmm_double(zeros_n, vlb_v2r, v2r_e, v2r_v, rule2,
                       vlb_v2v, v2v_e, v2v_v, x_j, rpw=rpw_n)
    return out[:N]


# speedup vs baseline: 1.3713x; 1.3713x over previous
import functools

import jax
import jax.numpy as jnp
from jax import lax
from jax.experimental import pallas as pl
from jax.experimental.pallas import tpu as pltpu
from jax.experimental.pallas import tpu_sc as plsc

NC = 2
NS = 16
NW = NC * NS
L = 16
CH = 128
D = 128

_EDGE_PAD = 2 * CH + 8


def _pad_edges(elist, value):
    z = jnp.zeros((_EDGE_PAD,), jnp.int32)
    zf = jnp.zeros((_EDGE_PAD,), jnp.float32)
    return jnp.concatenate([elist, z]), jnp.concatenate([value, zf])


def _vlb(vlist, n_rows, rpw):
    wb = ((rpw + 1 + 15) // 16) * 16
    idx = jnp.minimum(
        jnp.arange(NW, dtype=jnp.int32)[:, None] * rpw
        + jnp.arange(wb, dtype=jnp.int32)[None, :],
        n_rows,
    )
    return jnp.take(vlist, idx, axis=0).reshape(-1)


_GDN = lax.GatherDimensionNumbers(
    offset_dims=(), collapsed_slice_dims=(0,), start_index_map=(0,))


def _vbroadcast(v16, i):
    isp = jnp.full((L, 1), i, jnp.int32)
    return lax.gather(v16, isp, _GDN, (1,),
                      mode=lax.GatherScatterMode.PROMISE_IN_BOUNDS)


def _scalar(ref, i):
    v = ref[pl.ds((i // L) * L, L)]
    return v[i % L]


def _phase_runner(rpw, wb, nsteps, refs, wid, col_iota):
    (acc_v, gA, gB, vlb_v, idxA, idxB, valA, valB, rbA, rbB,
     semA, semB, semGA, semGB) = refs

    def search_chunk(base, e0, e1, val_v, rb_v):
        for g in range(CH // L):
            eid = jnp.full((L,), base + g * L, jnp.int32) + col_iota
            valid = (eid >= e0) & (eid < e1)
            v16 = val_v[pl.ds(g * L, L)]
            val_v[pl.ds(g * L, L)] = jnp.where(valid, v16, 0.0)
            lo = jnp.zeros((L,), jnp.int32)
            hi = jnp.full((L,), rpw, jnp.int32)
            for _s in range(nsteps):
                mid = (lo + hi) >> 1
                vm = plsc.load_gather(vlb_v, [mid])
                pred = vm <= eid
                lo = jnp.where(pred, mid, lo)
                hi = jnp.where(pred, hi, mid)
            rb_v[pl.ds(g * L, L)] = lo

    def edge_loop(g_v, val_v, rb_v):
        @plsc.parallel_loop(0, CH, unroll=8)
        def _(j):
            jsp = jnp.full((L,), j, jnp.int32)
            vj = plsc.load_gather(val_v, [jsp])
            rj = plsc.load_gather(rb_v, [jsp])
            for cb in range(D // L):
                ci = col_iota + (cb * L)
                g16 = plsc.load_gather(g_v, [jsp, ci])
                plsc.addupdate_scatter(acc_v, [rj, ci], vj * g16)

    def run_phase(vlb_hbm, el_hbm, val_hbm, src_hbm):
        pltpu.sync_copy(vlb_hbm.at[pl.ds(pl.multiple_of(wid * wb, 8), wb)],
                        vlb_v)
        e0 = _scalar(vlb_v, 0)
        e1 = _scalar(vlb_v, rpw)
        e0a = (e0 >> 3) << 3
        npair = (e1 - e0a + 2 * CH - 1) // (2 * CH)

        def pair_body(k, _):
            baseA = pl.multiple_of(e0a + k * (2 * CH), 8)
            baseB = pl.multiple_of(baseA + CH, 8)
            cA1 = pltpu.async_copy(el_hbm.at[pl.ds(baseA, CH)], idxA, semA)
            cA2 = pltpu.async_copy(val_hbm.at[pl.ds(baseA, CH)], valA, semA)
            cB1 = pltpu.async_copy(el_hbm.at[pl.ds(baseB, CH)], idxB, semB)
            cB2 = pltpu.async_copy(val_hbm.at[pl.ds(baseB, CH)], valB, semB)
            cA1.wait()
            cA2.wait()
            gcA = pltpu.async_copy(src_hbm.at[idxA], gA, semGA)
            search_chunk(baseA, e0, e1, valA, rbA)
            cB1.wait()
            cB2.wait()
            gcB = pltpu.async_copy(src_hbm.at[idxB], gB, semGB)
            search_chunk(baseB, e0, e1, valB, rbB)
            gcA.wait()
            edge_loop(gA, valA, rbA)
            gcB.wait()
            edge_loop(gB, valB, rbB)
            return 0

        lax.fori_loop(0, npair, pair_body, 0)

    return run_phase


def _scratch(rpw, wb):
    return [
        pltpu.VMEM((rpw, D), jnp.float32),
        pltpu.VMEM((CH, D), jnp.float32),
        pltpu.VMEM((CH, D), jnp.float32),
        pltpu.VMEM((wb,), jnp.int32),
        pltpu.VMEM((CH,), jnp.int32),
        pltpu.VMEM((CH,), jnp.int32),
        pltpu.VMEM((CH,), jnp.float32),
        pltpu.VMEM((CH,), jnp.float32),
        pltpu.VMEM((CH,), jnp.int32),
        pltpu.VMEM((CH,), jnp.int32),
        pltpu.SemaphoreType.DMA,
        pltpu.SemaphoreType.DMA,
        pltpu.SemaphoreType.DMA,
        pltpu.SemaphoreType.DMA,
    ]


_MESH = dict(core_axis_name="c", subcore_axis_name="s",
             num_cores=NC, num_subcores=NS)
_CP = pltpu.CompilerParams(needs_layout_passes=False)


@functools.partial(jax.jit, static_argnames=("rpw",))
def _spmm_single(accum, vlb_a, elist_a, value_a, src_a, *, rpw):
    n_rows_pad = NW * rpw
    wb = vlb_a.shape[0] // NW
    nsteps = max(1, (rpw - 1).bit_length())

    def body(accum_hbm, vlba_hbm, ela_hbm, vala_hbm, srca_hbm, out_hbm,
             *refs):
        wid = lax.axis_index("c") * NS + lax.axis_index("s")
        r0 = wid * rpw
        col_iota = lax.iota(jnp.int32, L)
        acc_v = refs[0]
        run = _phase_runner(rpw, wb, nsteps, refs, wid, col_iota)
        pltpu.sync_copy(accum_hbm.at[pl.ds(r0, rpw)], acc_v)
        run(vlba_hbm, ela_hbm, vala_hbm, srca_hbm)
        pltpu.sync_copy(acc_v, out_hbm.at[pl.ds(r0, rpw)])

    f = pl.kernel(
        body,
        out_type=jax.ShapeDtypeStruct((n_rows_pad, D), jnp.float32),
        mesh=plsc.VectorSubcoreMesh(**_MESH),
        scratch_types=_scratch(rpw, wb),
        compiler_params=_CP,
    )
    return f(accum, vlb_a, elist_a, value_a, src_a)


@functools.partial(jax.jit, static_argnames=("rpw",))
def _spmm_double(accum, vlb_a, elist_a, value_a, src_a,
                 vlb_b, elist_b, value_b, src_b, *, rpw):
    n_rows_pad = NW * rpw
    wb = vlb_a.shape[0] // NW
    nsteps = max(1, (rpw - 1).bit_length())

    def body(accum_hbm, vlba_hbm, ela_hbm, vala_hbm, srca_hbm,
             vlbb_hbm, elb_hbm, valb_hbm, srcb_hbm, out_hbm, *refs):
        wid = lax.axis_index("c") * NS + lax.axis_index("s")
        r0 = wid * rpw
        col_iota = lax.iota(jnp.int32, L)
        acc_v = refs[0]
        run = _phase_runner(rpw, wb, nsteps, refs, wid, col_iota)
        pltpu.sync_copy(accum_hbm.at[pl.ds(r0, rpw)], acc_v)
        run(vlba_hbm, ela_hbm, vala_hbm, srca_hbm)
        run(vlbb_hbm, elb_hbm, valb_hbm, srcb_hbm)
        pltpu.sync_copy(acc_v, out_hbm.at[pl.ds(r0, rpw)])

    f = pl.kernel(
        body,
        out_type=jax.ShapeDtypeStruct((n_rows_pad, D), jnp.float32),
        mesh=plsc.VectorSubcoreMesh(**_MESH),
        scratch_types=_scratch(rpw, wb),
        compiler_params=_CP,
    )
    return f(accum, vlb_a, elist_a, value_a, src_a,
             vlb_b, elist_b, value_b, src_b)


def kernel(x_j, v2v_vlist, v2v_elist, v2v_value,
           r2v_vlist, r2v_elist, r2v_value,
           r2r0_vlist, r2r0_elist, r2r0_value,
           r2r1_vlist, r2r1_elist, r2r1_value,
           v2r_vlist, v2r_elist, v2r_value):
    N = x_j.shape[0]
    R = r2v_vlist.shape[0] - 1

    rpw_r = -(-((R + NW - 1) // NW) // 8) * 8
    rpw_n = -(-((N + NW - 1) // NW) // 8) * 8
    rp_r = NW * rpw_r
    rp_n = NW * rpw_n

    r2v_e, r2v_v = _pad_edges(r2v_elist, r2v_value)
    r2r0_e, r2r0_v = _pad_edges(r2r0_elist, r2r0_value)
    r2r1_e, r2r1_v = _pad_edges(r2r1_elist, r2r1_value)
    v2r_e, v2r_v = _pad_edges(v2r_elist, v2r_value)
    v2v_e, v2v_v = _pad_edges(v2v_elist, v2v_value)

    vlb_r2v = _vlb(r2v_vlist, R, rpw_r)
    vlb_r2r0 = _vlb(r2r0_vlist, R, rpw_r)
    vlb_r2r1 = _vlb(r2r1_vlist, R, rpw_r)
    vlb_v2r = _vlb(v2r_vlist, N, rpw_n)
    vlb_v2v = _vlb(v2v_vlist, N, rpw_n)

    zeros_r = jnp.zeros((rp_r, D), jnp.float32)
    zeros_n = jnp.zeros((rp_n, D), jnp.float32)

    rule0 = _spmm_single(zeros_r, vlb_r2v, r2v_e, r2v_v, x_j, rpw=rpw_r)
    rule1 = _spmm_single(rule0, vlb_r2r0, r2r0_e, r2r0_v, rule0, rpw=rpw_r)
    rule2 = _spmm_single(rule1, vlb_r2r1, r2r1_e, r2r1_v, rule1, rpw=rpw_r)
    out = _spmm_double(zeros_n, vlb_v2r, v2r_e, v2r_v, rule2,
                       vlb_v2v, v2v_e, v2v_v, x_j, rpw=rpw_n)
    return out[:N]


# speedup vs baseline: 1.5746x; 1.1482x over previous
import functools

import jax
import jax.numpy as jnp
from jax import lax
from jax.experimental import pallas as pl
from jax.experimental.pallas import tpu as pltpu
from jax.experimental.pallas import tpu_sc as plsc

NC = 2
NS = 16
NW = NC * NS
L = 16
CH = 128
D = 128

_EDGE_PAD = 2 * CH + 8


def _pad_edges(elist, value):
    z = jnp.zeros((_EDGE_PAD,), jnp.int32)
    zf = jnp.zeros((_EDGE_PAD,), jnp.float32)
    return jnp.concatenate([elist, z]), jnp.concatenate([value, zf])


def _vlb(vlist, n_rows, rpw):
    wb = ((rpw + 1 + 15) // 16) * 16
    idx = jnp.minimum(
        jnp.arange(NW, dtype=jnp.int32)[:, None] * rpw
        + jnp.arange(wb, dtype=jnp.int32)[None, :],
        n_rows,
    )
    return jnp.take(vlist, idx, axis=0).reshape(-1)


_GDN = lax.GatherDimensionNumbers(
    offset_dims=(), collapsed_slice_dims=(0,), start_index_map=(0,))


def _vbroadcast(v16, i):
    isp = jnp.full((L, 1), i, jnp.int32)
    return lax.gather(v16, isp, _GDN, (1,),
                      mode=lax.GatherScatterMode.PROMISE_IN_BOUNDS)


def _scalar(ref, i):
    v = ref[pl.ds((i // L) * L, L)]
    return v[i % L]


def _phase_runner(rpw, wb, nsteps, refs, wid, soff, col_iota):
    (acc_v, gA, gB, vlb_v, idxA, idxB, valA, valB, rbA, rbB,
     semA, semB, semGA, semGB, semSA, semSB) = refs

    def search_chunk(base, e0, e1, val_v, rb_v):
        for g in range(CH // L):
            eid = jnp.full((L,), base + g * L, jnp.int32) + col_iota
            valid = (eid >= e0) & (eid < e1)
            v16 = val_v[pl.ds(g * L, L)]
            val_v[pl.ds(g * L, L)] = jnp.where(valid, v16, 0.0)
            lo = jnp.zeros((L,), jnp.int32)
            hi = jnp.full((L,), rpw, jnp.int32)
            for _s in range(nsteps):
                mid = (lo + hi) >> 1
                vm = plsc.load_gather(vlb_v, [mid])
                pred = vm <= eid
                lo = jnp.where(pred, mid, lo)
                hi = jnp.where(pred, hi, mid)
            rb_v[pl.ds(g * L, L)] = lo + soff

    def scale_chunk(g_v, val_v):
        @plsc.parallel_loop(0, CH, unroll=4)
        def _(j):
            jsp = jnp.full((L,), j, jnp.int32)
            vj = plsc.load_gather(val_v, [jsp])
            for cb in range(D // L):
                ci = col_iota + (cb * L)
                g16 = plsc.load_gather(g_v, [jsp, ci])
                plsc.store_scatter(g_v, [jsp, ci], vj * g16)

    def run_phase(vlb_hbm, el_hbm, val_hbm, src_hbm):
        pltpu.sync_copy(vlb_hbm.at[pl.ds(pl.multiple_of(wid * wb, 8), wb)],
                        vlb_v)
        e0 = _scalar(vlb_v, 0)
        e1 = _scalar(vlb_v, rpw)
        e0a = (e0 >> 3) << 3
        npair = (e1 - e0a + 2 * CH - 1) // (2 * CH)

        def wait_scatters():
            pltpu.make_async_copy(gA, acc_v.at[rbA], semSA).wait()
            pltpu.make_async_copy(gB, acc_v.at[rbB], semSB).wait()

        def pair_body(k, _):
            @pl.when(k > 0)
            def _():
                wait_scatters()

            baseA = pl.multiple_of(e0a + k * (2 * CH), 8)
            baseB = pl.multiple_of(baseA + CH, 8)
            cA1 = pltpu.async_copy(el_hbm.at[pl.ds(baseA, CH)], idxA, semA)
            cA2 = pltpu.async_copy(val_hbm.at[pl.ds(baseA, CH)], valA, semA)
            cB1 = pltpu.async_copy(el_hbm.at[pl.ds(baseB, CH)], idxB, semB)
            cB2 = pltpu.async_copy(val_hbm.at[pl.ds(baseB, CH)], valB, semB)
            cA1.wait()
            cA2.wait()
            gcA = pltpu.async_copy(src_hbm.at[idxA], gA, semGA)
            search_chunk(baseA, e0, e1, valA, rbA)
            cB1.wait()
            cB2.wait()
            gcB = pltpu.async_copy(src_hbm.at[idxB], gB, semGB)
            search_chunk(baseB, e0, e1, valB, rbB)
            gcA.wait()
            scale_chunk(gA, valA)
            pltpu.async_copy(gA, acc_v.at[rbA], semSA, add=True)
            gcB.wait()
            scale_chunk(gB, valB)
            pltpu.async_copy(gB, acc_v.at[rbB], semSB, add=True)
            return 0

        lax.fori_loop(0, npair, pair_body, 0)

        @pl.when(npair > 0)
        def _():
            wait_scatters()

    return run_phase


def _scratch(rpw, wb):
    return [
        pltpu.VMEM_SHARED((NS * rpw, D), jnp.float32),
        pltpu.VMEM((CH, D), jnp.float32),
        pltpu.VMEM((CH, D), jnp.float32),
        pltpu.VMEM((wb,), jnp.int32),
        pltpu.VMEM((CH,), jnp.int32),
        pltpu.VMEM((CH,), jnp.int32),
        pltpu.VMEM((CH,), jnp.float32),
        pltpu.VMEM((CH,), jnp.float32),
        pltpu.VMEM((CH,), jnp.int32),
        pltpu.VMEM((CH,), jnp.int32),
        pltpu.SemaphoreType.DMA,
        pltpu.SemaphoreType.DMA,
        pltpu.SemaphoreType.DMA,
        pltpu.SemaphoreType.DMA,
        pltpu.SemaphoreType.DMA,
        pltpu.SemaphoreType.DMA,
    ]


_MESH = dict(core_axis_name="c", subcore_axis_name="s",
             num_cores=NC, num_subcores=NS)
_CP = pltpu.CompilerParams(needs_layout_passes=False)


@functools.partial(jax.jit, static_argnames=("rpw",))
def _spmm_single(accum, vlb_a, elist_a, value_a, src_a, *, rpw):
    n_rows_pad = NW * rpw
    wb = vlb_a.shape[0] // NW
    nsteps = max(1, (rpw - 1).bit_length())

    def body(accum_hbm, vlba_hbm, ela_hbm, vala_hbm, srca_hbm, out_hbm,
             *refs):
        sid = lax.axis_index("s")
        wid = lax.axis_index("c") * NS + sid
        r0 = wid * rpw
        s0 = sid * rpw
        col_iota = lax.iota(jnp.int32, L)
        acc_v = refs[0]
        run = _phase_runner(rpw, wb, nsteps, refs, wid, s0, col_iota)
        pltpu.sync_copy(accum_hbm.at[pl.ds(r0, rpw)], acc_v.at[pl.ds(s0, rpw)])
        run(vlba_hbm, ela_hbm, vala_hbm, srca_hbm)
        pltpu.sync_copy(acc_v.at[pl.ds(s0, rpw)], out_hbm.at[pl.ds(r0, rpw)])

    f = pl.kernel(
        body,
        out_type=jax.ShapeDtypeStruct((n_rows_pad, D), jnp.float32),
        mesh=plsc.VectorSubcoreMesh(**_MESH),
        scratch_types=_scratch(rpw, wb),
        compiler_params=_CP,
    )
    return f(accum, vlb_a, elist_a, value_a, src_a)


@functools.partial(jax.jit, static_argnames=("rpw",))
def _spmm_double(accum, vlb_a, elist_a, value_a, src_a,
                 vlb_b, elist_b, value_b, src_b, *, rpw):
    n_rows_pad = NW * rpw
    wb = vlb_a.shape[0] // NW
    nsteps = max(1, (rpw - 1).bit_length())

    def body(accum_hbm, vlba_hbm, ela_hbm, vala_hbm, srca_hbm,
             vlbb_hbm, elb_hbm, valb_hbm, srcb_hbm, out_hbm, *refs):
        sid = lax.axis_index("s")
        wid = lax.axis_index("c") * NS + sid
        r0 = wid * rpw
        s0 = sid * rpw
        col_iota = lax.iota(jnp.int32, L)
        acc_v = refs[0]
        run = _phase_runner(rpw, wb, nsteps, refs, wid, s0, col_iota)
        pltpu.sync_copy(accum_hbm.at[pl.ds(r0, rpw)], acc_v.at[pl.ds(s0, rpw)])
        run(vlba_hbm, ela_hbm, vala_hbm, srca_hbm)
        run(vlbb_hbm, elb_hbm, valb_hbm, srcb_hbm)
        pltpu.sync_copy(acc_v.at[pl.ds(s0, rpw)], out_hbm.at[pl.ds(r0, rpw)])

    f = pl.kernel(
        body,
        out_type=jax.ShapeDtypeStruct((n_rows_pad, D), jnp.float32),
        mesh=plsc.VectorSubcoreMesh(**_MESH),
        scratch_types=_scratch(rpw, wb),
        compiler_params=_CP,
    )
    return f(accum, vlb_a, elist_a, value_a, src_a,
             vlb_b, elist_b, value_b, src_b)


def kernel(x_j, v2v_vlist, v2v_elist, v2v_value,
           r2v_vlist, r2v_elist, r2v_value,
           r2r0_vlist, r2r0_elist, r2r0_value,
           r2r1_vlist, r2r1_elist, r2r1_value,
           v2r_vlist, v2r_elist, v2r_value):
    N = x_j.shape[0]
    R = r2v_vlist.shape[0] - 1

    rpw_r = -(-((R + NW - 1) // NW) // 8) * 8
    rpw_n = -(-((N + NW - 1) // NW) // 8) * 8
    rp_r = NW * rpw_r
    rp_n = NW * rpw_n

    r2v_e, r2v_v = _pad_edges(r2v_elist, r2v_value)
    r2r0_e, r2r0_v = _pad_edges(r2r0_elist, r2r0_value)
    r2r1_e, r2r1_v = _pad_edges(r2r1_elist, r2r1_value)
    v2r_e, v2r_v = _pad_edges(v2r_elist, v2r_value)
    v2v_e, v2v_v = _pad_edges(v2v_elist, v2v_value)

    vlb_r2v = _vlb(r2v_vlist, R, rpw_r)
    vlb_r2r0 = _vlb(r2r0_vlist, R, rpw_r)
    vlb_r2r1 = _vlb(r2r1_vlist, R, rpw_r)
    vlb_v2r = _vlb(v2r_vlist, N, rpw_n)
    vlb_v2v = _vlb(v2v_vlist, N, rpw_n)

    zeros_r = jnp.zeros((rp_r, D), jnp.float32)
    zeros_n = jnp.zeros((rp_n, D), jnp.float32)

    rule0 = _spmm_single(zeros_r, vlb_r2v, r2v_e, r2v_v, x_j, rpw=rpw_r)
    rule1 = _spmm_single(rule0, vlb_r2r0, r2r0_e, r2r0_v, rule0, rpw=rpw_r)
    rule2 = _spmm_single(rule1, vlb_r2r1, r2r1_e, r2r1_v, rule1, rpw=rpw_r)
    out = _spmm_double(zeros_n, vlb_v2r, v2r_e, v2r_v, rule2,
                       vlb_v2v, v2v_e, v2v_v, x_j, rpw=rpw_n)
    return out[:N]


# speedup vs baseline: 1.7671x; 1.1223x over previous
import functools

import jax
import jax.numpy as jnp
from jax import lax
from jax.experimental import pallas as pl
from jax.experimental.pallas import tpu as pltpu
from jax.experimental.pallas import tpu_sc as plsc

NC = 2
NS = 16
NW = NC * NS
L = 16
CH = 128
D = 128

_EDGE_PAD = 2 * CH + 8


def _pad_edges(elist, value):
    z = jnp.zeros((_EDGE_PAD,), jnp.int32)
    zf = jnp.zeros((_EDGE_PAD,), jnp.float32)
    return jnp.concatenate([elist, z]), jnp.concatenate([value, zf])


def _vlb(vlist, n_rows, rpw):
    wb = ((rpw + 1 + 15) // 16) * 16
    idx = jnp.minimum(
        jnp.arange(NW, dtype=jnp.int32)[:, None] * rpw
        + jnp.arange(wb, dtype=jnp.int32)[None, :],
        n_rows,
    )
    return jnp.take(vlist, idx, axis=0).reshape(-1)


_GDN = lax.GatherDimensionNumbers(
    offset_dims=(), collapsed_slice_dims=(0,), start_index_map=(0,))


def _vbroadcast(v16, i):
    isp = jnp.full((L, 1), i, jnp.int32)
    return lax.gather(v16, isp, _GDN, (1,),
                      mode=lax.GatherScatterMode.PROMISE_IN_BOUNDS)


def _scalar(ref, i):
    v = ref[pl.ds((i // L) * L, L)]
    return v[i % L]


def _phase_runner(rpw, wb, nsteps, refs, wid, soff, col_iota):
    (acc_v, gA, gB, vlb_v, idxA, idxB, valA, valB, rbA, rbB,
     semA, semB, semGA, semGB, semSA, semSB) = refs

    def search_chunk(base, e0, e1, val_v, rb_v):
        for g in range(CH // L):
            eid = jnp.full((L,), base + g * L, jnp.int32) + col_iota
            valid = (eid >= e0) & (eid < e1)
            v16 = val_v[pl.ds(g * L, L)]
            val_v[pl.ds(g * L, L)] = jnp.where(valid, v16, 0.0)
            lo = jnp.zeros((L,), jnp.int32)
            hi = jnp.full((L,), rpw, jnp.int32)
            for _s in range(nsteps):
                mid = (lo + hi) >> 1
                vm = plsc.load_gather(vlb_v, [mid])
                pred = vm <= eid
                lo = jnp.where(pred, mid, lo)
                hi = jnp.where(pred, hi, mid)
            rb_v[pl.ds(g * L, L)] = lo + soff

    def scale_chunk(g_v, val_v):
        @plsc.parallel_loop(0, CH, unroll=4)
        def _(j):
            jsp = jnp.full((L,), j, jnp.int32)
            vj = plsc.load_gather(val_v, [jsp])
            for cb in range(D // L):
                sl = (j, pl.ds(cb * L, L))
                g_v[sl] = g_v[sl] * vj

    def run_phase(vlb_hbm, el_hbm, val_hbm, src_hbm):
        pltpu.sync_copy(vlb_hbm.at[pl.ds(pl.multiple_of(wid * wb, 8), wb)],
                        vlb_v)
        e0 = _scalar(vlb_v, 0)
        e1 = _scalar(vlb_v, rpw)
        e0a = (e0 >> 3) << 3
        npair = (e1 - e0a + 2 * CH - 1) // (2 * CH)

        def wait_scatters():
            pltpu.make_async_copy(gA, acc_v.at[rbA], semSA).wait()
            pltpu.make_async_copy(gB, acc_v.at[rbB], semSB).wait()

        def pair_body(k, _):
            @pl.when(k > 0)
            def _():
                wait_scatters()

            baseA = pl.multiple_of(e0a + k * (2 * CH), 8)
            baseB = pl.multiple_of(baseA + CH, 8)
            cA1 = pltpu.async_copy(el_hbm.at[pl.ds(baseA, CH)], idxA, semA)
            cA2 = pltpu.async_copy(val_hbm.at[pl.ds(baseA, CH)], valA, semA)
            cB1 = pltpu.async_copy(el_hbm.at[pl.ds(baseB, CH)], idxB, semB)
            cB2 = pltpu.async_copy(val_hbm.at[pl.ds(baseB, CH)], valB, semB)
            cA1.wait()
            cA2.wait()
            gcA = pltpu.async_copy(src_hbm.at[idxA], gA, semGA)
            search_chunk(baseA, e0, e1, valA, rbA)
            cB1.wait()
            cB2.wait()
            gcB = pltpu.async_copy(src_hbm.at[idxB], gB, semGB)
            search_chunk(baseB, e0, e1, valB, rbB)
            gcA.wait()
            scale_chunk(gA, valA)
            pltpu.async_copy(gA, acc_v.at[rbA], semSA, add=True)
            gcB.wait()
            scale_chunk(gB, valB)
            pltpu.async_copy(gB, acc_v.at[rbB], semSB, add=True)
            return 0

        lax.fori_loop(0, npair, pair_body, 0)

        @pl.when(npair > 0)
        def _():
            wait_scatters()

    return run_phase


def _scratch(rpw, wb):
    return [
        pltpu.VMEM_SHARED((NS * rpw, D), jnp.float32),
        pltpu.VMEM((CH, D), jnp.float32),
        pltpu.VMEM((CH, D), jnp.float32),
        pltpu.VMEM((wb,), jnp.int32),
        pltpu.VMEM((CH,), jnp.int32),
        pltpu.VMEM((CH,), jnp.int32),
        pltpu.VMEM((CH,), jnp.float32),
        pltpu.VMEM((CH,), jnp.float32),
        pltpu.VMEM((CH,), jnp.int32),
        pltpu.VMEM((CH,), jnp.int32),
        pltpu.SemaphoreType.DMA,
        pltpu.SemaphoreType.DMA,
        pltpu.SemaphoreType.DMA,
        pltpu.SemaphoreType.DMA,
        pltpu.SemaphoreType.DMA,
        pltpu.SemaphoreType.DMA,
    ]


_MESH = dict(core_axis_name="c", subcore_axis_name="s",
             num_cores=NC, num_subcores=NS)
_CP = pltpu.CompilerParams(needs_layout_passes=False)


@functools.partial(jax.jit, static_argnames=("rpw",))
def _spmm_single(accum, vlb_a, elist_a, value_a, src_a, *, rpw):
    n_rows_pad = NW * rpw
    wb = vlb_a.shape[0] // NW
    nsteps = max(1, (rpw - 1).bit_length())

    def body(accum_hbm, vlba_hbm, ela_hbm, vala_hbm, srca_hbm, out_hbm,
             *refs):
        sid = lax.axis_index("s")
        wid = lax.axis_index("c") * NS + sid
        r0 = wid * rpw
        s0 = sid * rpw
        col_iota = lax.iota(jnp.int32, L)
        acc_v = refs[0]
        run = _phase_runner(rpw, wb, nsteps, refs, wid, s0, col_iota)
        pltpu.sync_copy(accum_hbm.at[pl.ds(r0, rpw)], acc_v.at[pl.ds(s0, rpw)])
        run(vlba_hbm, ela_hbm, vala_hbm, srca_hbm)
        pltpu.sync_copy(acc_v.at[pl.ds(s0, rpw)], out_hbm.at[pl.ds(r0, rpw)])

    f = pl.kernel(
        body,
        out_type=jax.ShapeDtypeStruct((n_rows_pad, D), jnp.float32),
        mesh=plsc.VectorSubcoreMesh(**_MESH),
        scratch_types=_scratch(rpw, wb),
        compiler_params=_CP,
    )
    return f(accum, vlb_a, elist_a, value_a, src_a)


@functools.partial(jax.jit, static_argnames=("rpw",))
def _spmm_double(accum, vlb_a, elist_a, value_a, src_a,
                 vlb_b, elist_b, value_b, src_b, *, rpw):
    n_rows_pad = NW * rpw
    wb = vlb_a.shape[0] // NW
    nsteps = max(1, (rpw - 1).bit_length())

    def body(accum_hbm, vlba_hbm, ela_hbm, vala_hbm, srca_hbm,
             vlbb_hbm, elb_hbm, valb_hbm, srcb_hbm, out_hbm, *refs):
        sid = lax.axis_index("s")
        wid = lax.axis_index("c") * NS + sid
        r0 = wid * rpw
        s0 = sid * rpw
        col_iota = lax.iota(jnp.int32, L)
        acc_v = refs[0]
        run = _phase_runner(rpw, wb, nsteps, refs, wid, s0, col_iota)
        pltpu.sync_copy(accum_hbm.at[pl.ds(r0, rpw)], acc_v.at[pl.ds(s0, rpw)])
        run(vlba_hbm, ela_hbm, vala_hbm, srca_hbm)
        run(vlbb_hbm, elb_hbm, valb_hbm, srcb_hbm)
        pltpu.sync_copy(acc_v.at[pl.ds(s0, rpw)], out_hbm.at[pl.ds(r0, rpw)])

    f = pl.kernel(
        body,
        out_type=jax.ShapeDtypeStruct((n_rows_pad, D), jnp.float32),
        mesh=plsc.VectorSubcoreMesh(**_MESH),
        scratch_types=_scratch(rpw, wb),
        compiler_params=_CP,
    )
    return f(accum, vlb_a, elist_a, value_a, src_a,
             vlb_b, elist_b, value_b, src_b)


def kernel(x_j, v2v_vlist, v2v_elist, v2v_value,
           r2v_vlist, r2v_elist, r2v_value,
           r2r0_vlist, r2r0_elist, r2r0_value,
           r2r1_vlist, r2r1_elist, r2r1_value,
           v2r_vlist, v2r_elist, v2r_value):
    N = x_j.shape[0]
    R = r2v_vlist.shape[0] - 1

    rpw_r = -(-((R + NW - 1) // NW) // 8) * 8
    rpw_n = -(-((N + NW - 1) // NW) // 8) * 8
    rp_r = NW * rpw_r
    rp_n = NW * rpw_n

    r2v_e, r2v_v = _pad_edges(r2v_elist, r2v_value)
    r2r0_e, r2r0_v = _pad_edges(r2r0_elist, r2r0_value)
    r2r1_e, r2r1_v = _pad_edges(r2r1_elist, r2r1_value)
    v2r_e, v2r_v = _pad_edges(v2r_elist, v2r_value)
    v2v_e, v2v_v = _pad_edges(v2v_elist, v2v_value)

    vlb_r2v = _vlb(r2v_vlist, R, rpw_r)
    vlb_r2r0 = _vlb(r2r0_vlist, R, rpw_r)
    vlb_r2r1 = _vlb(r2r1_vlist, R, rpw_r)
    vlb_v2r = _vlb(v2r_vlist, N, rpw_n)
    vlb_v2v = _vlb(v2v_vlist, N, rpw_n)

    zeros_r = jnp.zeros((rp_r, D), jnp.float32)
    zeros_n = jnp.zeros((rp_n, D), jnp.float32)

    rule0 = _spmm_single(zeros_r, vlb_r2v, r2v_e, r2v_v, x_j, rpw=rpw_r)
    rule1 = _spmm_single(rule0, vlb_r2r0, r2r0_e, r2r0_v, rule0, rpw=rpw_r)
    rule2 = _spmm_single(rule1, vlb_r2r1, r2r1_e, r2r1_v, rule1, rpw=rpw_r)
    out = _spmm_double(zeros_n, vlb_v2r, v2r_e, v2r_v, rule2,
                       vlb_v2v, v2v_e, v2v_v, x_j, rpw=rpw_n)
    return out[:N]
